# baseline (device time: 54430 ns/iter reference)
import jax
import jax.numpy as jnp
from jax import lax
from jax.experimental import pallas as pl
from jax.experimental.pallas import tpu as pltpu

N_DEV = 32
SQ = 256
D = 1024
SKV = 4096
DH = 128
H_LOCAL = 8
KV_LOCAL = 2
SCALE = 0.08838834764831843

MESH = pl.DeviceIdType.MESH

_CR = 0
_CA = 256
_SEM_RS = 0
_SEM_AG = 31
N_SEMS = 62


def _body(x_ref, wq_ref, wo_ref, k_hbm, v_hbm, out_ref,
          k_vmem, v_vmem, cp_sems, comm_ref, stage_ref,
          send_sems, recv_sems):
    m = lax.axis_index("i")

    barrier_sem = pltpu.get_barrier_semaphore()
    for d in range(1, N_DEV):
        pl.semaphore_signal(barrier_sem, inc=1, device_id=((m + d) % N_DEV,),
                            device_id_type=MESH)
    pl.semaphore_wait(barrier_sem, N_DEV - 1)

    kv0 = m * KV_LOCAL
    k_cp = pltpu.make_async_copy(
        k_hbm.at[0, :, pl.ds(kv0, KV_LOCAL), :], k_vmem, cp_sems.at[0])
    v_cp = pltpu.make_async_copy(
        v_hbm.at[0, :, pl.ds(kv0, KV_LOCAL), :], v_vmem, cp_sems.at[1])
    k_cp.start()
    v_cp.start()

    q = jnp.dot(x_ref[0], wq_ref[...], preferred_element_type=jnp.float32)

    k_cp.wait()
    v_cp.wait()

    kb = k_vmem[...].astype(jnp.bfloat16)
    qb = q.astype(jnp.bfloat16)

    outs = []
    for h in range(H_LOCAL):
        q_h = qb[:, h * DH:(h + 1) * DH]
        k_h = kb[:, h // 4, :]
        v_h = v_vmem[:, h // 4, :]
        s = lax.dot_general(
            q_h, k_h, (((1,), (1,)), ((), ())),
            preferred_element_type=jnp.float32) * SCALE
        mx = jnp.max(s, axis=1, keepdims=True)
        p = jnp.exp(s - mx)
        l = jnp.sum(p, axis=1, keepdims=True)
        o_h = jnp.dot(p, v_h, preferred_element_type=jnp.float32) / l
        outs.append(o_h)
    attn = jnp.concatenate(outs, axis=1)

    out_ref[0, :, :] = jnp.dot(attn, wo_ref[...],
                               preferred_element_type=jnp.float32)

    f32 = jnp.float32
    bf16 = jnp.bfloat16
    myoff = pl.multiple_of(8 * m, 8)

    stage_ref[0:SQ, :] = out_ref[0, :, :].astype(bf16)
    rs = []
    for d in range(1, N_DEV):
        p = (m + d) % N_DEV
        r = pltpu.make_async_remote_copy(
            src_ref=stage_ref.at[pl.ds(pl.multiple_of(8 * p, 8), 8), :],
            dst_ref=comm_ref.at[pl.ds(pl.multiple_of(_CR + 8 * m, 8), 8), :],
            send_sem=send_sems.at[_SEM_RS + d - 1],
            recv_sem=recv_sems.at[_SEM_RS + d - 1],
            device_id=(p,),
            device_id_type=MESH,
        )
        r.start()
        rs.append(r)
    comm_ref[pl.ds(pl.multiple_of(_CR + myoff, 8), 8), :] = (
        stage_ref[pl.ds(myoff, 8), :])
    for r in rs:
        r.wait_send()
    for r in rs:
        r.wait_recv()
    reduced = jnp.sum(
        comm_ref[_CR:_CR + SQ, :].astype(f32).reshape(N_DEV, 8, D), axis=0)
    out_ref[0, pl.ds(myoff, 8), :] = reduced

    stage_ref[0:8, :] = reduced.astype(bf16)
    ag = []
    for d in range(1, N_DEV):
        p = (m + d) % N_DEV
        r = pltpu.make_async_remote_copy(
            src_ref=stage_ref.at[0:8, :],
            dst_ref=comm_ref.at[pl.ds(pl.multiple_of(_CA + 8 * m, 8), 8), :],
            send_sem=send_sems.at[_SEM_AG + d - 1],
            recv_sem=recv_sems.at[_SEM_AG + d - 1],
            device_id=(p,),
            device_id_type=MESH,
        )
        r.start()
        ag.append(r)
    comm_ref[pl.ds(pl.multiple_of(_CA + myoff, 8), 8), :] = stage_ref[0:8, :]
    for r in ag:
        r.wait_send()
    for r in ag:
        r.wait_recv()
    out_ref[0, :, :] = comm_ref[_CA:_CA + SQ, :].astype(f32)


def kernel(x, Wq, Wo, K_ext, V_ext):
    return pl.pallas_call(
        _body,
        out_shape=jax.ShapeDtypeStruct((1, SQ, D), jnp.float32),
        in_specs=[
            pl.BlockSpec(memory_space=pltpu.VMEM),
            pl.BlockSpec(memory_space=pltpu.VMEM),
            pl.BlockSpec(memory_space=pltpu.VMEM),
            pl.BlockSpec(memory_space=pl.ANY),
            pl.BlockSpec(memory_space=pl.ANY),
        ],
        out_specs=pl.BlockSpec(memory_space=pltpu.VMEM),
        scratch_shapes=[
            pltpu.VMEM((SKV, KV_LOCAL, DH), jnp.float32),
            pltpu.VMEM((SKV, KV_LOCAL, DH), jnp.float32),
            pltpu.SemaphoreType.DMA((2,)),
            pltpu.VMEM((512, D), jnp.bfloat16),
            pltpu.VMEM((SQ, D), jnp.bfloat16),
            pltpu.SemaphoreType.DMA((N_SEMS,)),
            pltpu.SemaphoreType.DMA((N_SEMS,)),
        ],
        compiler_params=pltpu.CompilerParams(
            collective_id=0,
            vmem_limit_bytes=100 * 1024 * 1024),
    )(x, Wq, Wo, K_ext, V_ext)


# device time: 53288 ns/iter; 1.0214x vs baseline; 1.0214x over previous
import jax
import jax.numpy as jnp
from jax import lax
from jax.experimental import pallas as pl
from jax.experimental.pallas import tpu as pltpu

N_DEV = 32
SQ = 256
D = 1024
SKV = 4096
DH = 128
H_LOCAL = 8
KV_LOCAL = 2
SCALE = 0.08838834764831843

MESH = pl.DeviceIdType.MESH

_CR = 0
_CA = 256
_SEM_RS = 0
_SEM_AG = 31
N_SEMS = 62


def _body(x_ref, wq_ref, wo_ref, k_hbm, v_hbm, out_ref,
          k_vmem, v_vmem, cp_sems, comm_ref, stage_ref,
          send_sems, recv_sems):
    m = lax.axis_index("i")

    barrier_sem = pltpu.get_barrier_semaphore()
    for d in range(1, N_DEV):
        pl.semaphore_signal(barrier_sem, inc=1, device_id=((m + d) % N_DEV,),
                            device_id_type=MESH)
    pl.semaphore_wait(barrier_sem, N_DEV - 1)

    kv0 = m * KV_LOCAL
    k_cp = pltpu.make_async_copy(
        k_hbm.at[0, :, pl.ds(kv0, KV_LOCAL), :], k_vmem, cp_sems.at[0])
    v_cp = pltpu.make_async_copy(
        v_hbm.at[0, :, pl.ds(kv0, KV_LOCAL), :], v_vmem, cp_sems.at[1])
    k_cp.start()
    v_cp.start()

    q = jnp.dot(x_ref[0], wq_ref[...], preferred_element_type=jnp.float32)

    k_cp.wait()
    v_cp.wait()

    outs = []
    for h in range(H_LOCAL):
        q_h = q[:, h * DH:(h + 1) * DH]
        k_h = k_vmem[:, h // 4, :]
        v_h = v_vmem[:, h // 4, :]
        s = lax.dot_general(
            q_h, k_h, (((1,), (1,)), ((), ())),
            preferred_element_type=jnp.float32) * SCALE
        mx = jnp.max(s, axis=1, keepdims=True)
        p = jnp.exp(s - mx)
        l = jnp.sum(p, axis=1, keepdims=True)
        o_h = jnp.dot(p, v_h, preferred_element_type=jnp.float32) / l
        outs.append(o_h)
    attn = jnp.concatenate(outs, axis=1)

    out_ref[0, :, :] = jnp.dot(attn, wo_ref[...],
                               preferred_element_type=jnp.float32)

    f32 = jnp.float32
    bf16 = jnp.bfloat16
    myoff = pl.multiple_of(8 * m, 8)

    stage_ref[0:SQ, :] = out_ref[0, :, :].astype(bf16)
    rs = []
    for d in range(1, N_DEV):
        p = (m + d) % N_DEV
        r = pltpu.make_async_remote_copy(
            src_ref=stage_ref.at[pl.ds(pl.multiple_of(8 * p, 8), 8), :],
            dst_ref=comm_ref.at[pl.ds(pl.multiple_of(_CR + 8 * m, 8), 8), :],
            send_sem=send_sems.at[_SEM_RS + d - 1],
            recv_sem=recv_sems.at[_SEM_RS + d - 1],
            device_id=(p,),
            device_id_type=MESH,
        )
        r.start()
        rs.append(r)
    comm_ref[pl.ds(pl.multiple_of(_CR + myoff, 8), 8), :] = (
        stage_ref[pl.ds(myoff, 8), :])
    for r in rs:
        r.wait_send()
    for r in rs:
        r.wait_recv()
    reduced = jnp.sum(
        comm_ref[_CR:_CR + SQ, :].astype(f32).reshape(N_DEV, 8, D), axis=0)
    out_ref[0, pl.ds(myoff, 8), :] = reduced

    stage_ref[0:8, :] = reduced.astype(bf16)
    ag = []
    for d in range(1, N_DEV):
        p = (m + d) % N_DEV
        r = pltpu.make_async_remote_copy(
            src_ref=stage_ref.at[0:8, :],
            dst_ref=comm_ref.at[pl.ds(pl.multiple_of(_CA + 8 * m, 8), 8), :],
            send_sem=send_sems.at[_SEM_AG + d - 1],
            recv_sem=recv_sems.at[_SEM_AG + d - 1],
            device_id=(p,),
            device_id_type=MESH,
        )
        r.start()
        ag.append(r)
    comm_ref[pl.ds(pl.multiple_of(_CA + myoff, 8), 8), :] = stage_ref[0:8, :]
    for r in ag:
        r.wait_send()
    for r in ag:
        r.wait_recv()
    out_ref[0, :, :] = comm_ref[_CA:_CA + SQ, :].astype(f32)


def kernel(x, Wq, Wo, K_ext, V_ext):
    return pl.pallas_call(
        _body,
        out_shape=jax.ShapeDtypeStruct((1, SQ, D), jnp.float32),
        in_specs=[
            pl.BlockSpec(memory_space=pltpu.VMEM),
            pl.BlockSpec(memory_space=pltpu.VMEM),
            pl.BlockSpec(memory_space=pltpu.VMEM),
            pl.BlockSpec(memory_space=pl.ANY),
            pl.BlockSpec(memory_space=pl.ANY),
        ],
        out_specs=pl.BlockSpec(memory_space=pltpu.VMEM),
        scratch_shapes=[
            pltpu.VMEM((SKV, KV_LOCAL, DH), jnp.float32),
            pltpu.VMEM((SKV, KV_LOCAL, DH), jnp.float32),
            pltpu.SemaphoreType.DMA((2,)),
            pltpu.VMEM((512, D), jnp.bfloat16),
            pltpu.VMEM((SQ, D), jnp.bfloat16),
            pltpu.SemaphoreType.DMA((N_SEMS,)),
            pltpu.SemaphoreType.DMA((N_SEMS,)),
        ],
        compiler_params=pltpu.CompilerParams(
            collective_id=0,
            vmem_limit_bytes=100 * 1024 * 1024),
    )(x, Wq, Wo, K_ext, V_ext)
